# Initial kernel scaffold; baseline (speedup 1.0000x reference)
import jax
import jax.numpy as jnp
from jax.experimental import pallas as pl


def kernel(input_ids, embed_tokens):
    raise NotImplementedError("write your pallas kernel here")


# speedup vs baseline: 6.6287x; 6.6287x over previous
import functools

import jax
import jax.numpy as jnp
from jax import lax
from jax.experimental import pallas as pl
from jax.experimental.pallas import tpu as pltpu
from jax.experimental.pallas import tpu_sc as plsc

R, C, D = 16384, 200, 4
NC, NS, L = 2, 16, 16
NW = NC * NS
ROWS_W = R // NW
CH = 32
NCHUNK = ROWS_W // CH
GROUPS = (C * D) // L


def _compute_chunk(ids_ref, out_ref, tab_ref, cpat, dpat):
    def row_body(rr, carry):
        rspl = jnp.full((L,), rr, dtype=jnp.int32)

        def grp_body(k, carry2):
            cv = cpat + k * 4
            idv = plsc.load_gather(ids_ref, [rspl, cv])
            val = plsc.load_gather(tab_ref, [idv, dpat]) + jnp.float32(0.02)
            plsc.store_scatter(out_ref, [rspl, cv, dpat], val)
            return carry2

        return lax.fori_loop(0, GROUPS, grp_body, carry)

    lax.fori_loop(0, CH, row_body, 0)


def _body(ids_hbm, tab_hbm, out_hbm,
          tab_v, ids_v0, ids_v1, out_v0, out_v1,
          in_s0, in_s1, out_s0, out_s1):
    wid = lax.axis_index("s") * NC + lax.axis_index("c")
    row0 = wid * ROWS_W

    pltpu.sync_copy(tab_hbm, tab_v)
    iot = lax.iota(jnp.int32, L)
    cpat = lax.shift_right_logical(iot, 2)
    dpat = lax.bitwise_and(iot, 3)

    ids_bufs = (ids_v0, ids_v1)
    out_bufs = (out_v0, out_v1)
    in_sems = (in_s0, in_s1)
    out_sems = (out_s0, out_s1)

    in_cp = {}
    out_cp = {}
    in_cp[0] = pltpu.async_copy(
        ids_hbm.at[pl.ds(row0, CH)], ids_bufs[0], in_sems[0])
    for g in range(NCHUNK):
        b = g & 1
        in_cp[g].wait()
        if g + 1 < NCHUNK:
            in_cp[g + 1] = pltpu.async_copy(
                ids_hbm.at[pl.ds(row0 + (g + 1) * CH, CH)],
                ids_bufs[1 - b], in_sems[1 - b])
        if g >= 2:
            out_cp[g - 2].wait()
        _compute_chunk(ids_bufs[b], out_bufs[b], tab_v, cpat, dpat)
        out_cp[g] = pltpu.async_copy(
            out_bufs[b], out_hbm.at[pl.ds(row0 + g * CH, CH)], out_sems[b])
    out_cp[NCHUNK - 2].wait()
    out_cp[NCHUNK - 1].wait()


_sc_lookup = functools.partial(
    pl.kernel,
    out_type=jax.ShapeDtypeStruct((R, C, D), jnp.float32),
    mesh=plsc.VectorSubcoreMesh(
        core_axis_name="c", subcore_axis_name="s",
        num_cores=NC, num_subcores=NS),
    scratch_types=[
        pltpu.VMEM((8, D), jnp.float32),
        pltpu.VMEM((CH, C), jnp.int32),
        pltpu.VMEM((CH, C), jnp.int32),
        pltpu.VMEM((CH, C, D), jnp.float32),
        pltpu.VMEM((CH, C, D), jnp.float32),
        pltpu.SemaphoreType.DMA,
        pltpu.SemaphoreType.DMA,
        pltpu.SemaphoreType.DMA,
        pltpu.SemaphoreType.DMA,
    ],
    compiler_params=pltpu.CompilerParams(
        use_tc_tiling_on_sc=False, needs_layout_passes=False),
)(_body)


def kernel(input_ids, embed_tokens):
    return _sc_lookup(input_ids.astype(jnp.int32),
                      embed_tokens.astype(jnp.float32))
